# Initial kernel scaffold; baseline (speedup 1.0000x reference)
import functools

import jax
import jax.numpy as jnp
from jax import lax
from jax.experimental import pallas as pl
from jax.experimental.pallas import tpu as pltpu


_BLK = 500


def _lin_body(nout, x_ref, *refs):
    x = x_ref[...]
    for i in range(nout):
        W = refs[i][...]
        b = refs[nout + i][...]
        o = refs[2 * nout + i]
        o[...] = lax.dot_general(
            x, W, (((1,), (1,)), ((), ())),
            preferred_element_type=jnp.float32,
            precision=lax.Precision.HIGHEST,
        ) + b


def _linears(x, Ws, bs):
    n, d = x.shape
    nout = len(Ws)
    grid = (n + _BLK - 1) // _BLK
    out_shapes = [jax.ShapeDtypeStruct((n, Ws[i].shape[0]), jnp.float32)
                  for i in range(nout)]
    in_specs = [pl.BlockSpec((_BLK, d), lambda i: (i, 0))]
    for W in Ws:
        in_specs.append(pl.BlockSpec(W.shape, lambda i: (0, 0)))
    for b in bs:
        in_specs.append(pl.BlockSpec((1, b.shape[1]), lambda i: (0, 0)))
    out_specs = [pl.BlockSpec((_BLK, Ws[i].shape[0]), lambda i: (i, 0))
                 for i in range(nout)]
    return pl.pallas_call(
        functools.partial(_lin_body, nout),
        grid=grid,
        in_specs=in_specs,
        out_specs=out_specs,
        out_shape=out_shapes,
    )(x, *Ws, *bs)


def kernel(feat_word, feat_topic, effect, src_ww, dst_ww, ew_ww, src_wt, dst_wt, ew_wt, src_wd, dst_wd, ew_wd, src_td, dst_td, ew_td, src_tt, dst_tt, ew_tt, W_ww, b_ww, W_wt, b_wt, W_wd, b_wd, W_td, b_td, W_tt, b_tt, W_td_cau, b_td_cau, W_td_noi, b_td_noi, W_tt_cau, b_tt_cau, W_tt_noi, b_tt_noi, W_td_cau_trans, W_td_noi_trans, W_tt_cau_trans, W_tt_noi_trans):
    Nw, D = feat_word.shape
    Nt = feat_topic.shape[0]
    Nd = 50000

    Wh_ww, Wh_wt, Wh_wd = _linears(
        feat_word, [W_ww, W_wt, W_wd],
        [b_ww.reshape(1, -1), b_wt.reshape(1, -1), b_wd.reshape(1, -1)])
    Wh_td, Wh_tt = _linears(
        feat_topic, [W_td, W_tt],
        [b_td.reshape(1, -1), b_tt.reshape(1, -1)])

    def agg(Wh, src, dst, ew, nd):
        m = jnp.take(Wh, src, axis=0) * ew
        msum = jax.ops.segment_sum(m, dst, num_segments=nd)
        deg = jax.ops.segment_sum(jnp.ones((dst.shape[0], 1), jnp.float32),
                                  dst, num_segments=nd)
        return msum / jnp.maximum(deg, 1.0)

    h_word = agg(Wh_ww, src_ww, dst_ww, ew_ww, Nw)
    h_topic = (agg(Wh_wt, src_wt, dst_wt, ew_wt, Nt)
               + agg(Wh_tt, src_tt, dst_tt, ew_tt, Nt))
    h_doc = (agg(Wh_wd, src_wd, dst_wd, ew_wd, Nd)
             + agg(Wh_td, src_td, dst_td, ew_td, Nd))
    return (h_word, h_topic, h_doc)


# speedup vs baseline: 1.0518x; 1.0518x over previous
import functools

import jax
import jax.numpy as jnp
from jax import lax
from jax.experimental import pallas as pl
from jax.experimental.pallas import tpu as pltpu


_BLK = 1000


def _lin_body(nout, x_ref, *refs):
    x = x_ref[...]
    for i in range(nout):
        W = refs[i][...]
        b = refs[nout + i][...]
        o = refs[2 * nout + i]
        o[...] = lax.dot_general(
            x, W, (((1,), (1,)), ((), ())),
            preferred_element_type=jnp.float32,
            precision=lax.Precision.HIGHEST,
        ) + b


def _linears(x, Ws, bs):
    n, d = x.shape
    nout = len(Ws)
    grid = (n + _BLK - 1) // _BLK
    out_shapes = [jax.ShapeDtypeStruct((n, Ws[i].shape[0]), jnp.float32)
                  for i in range(nout)]
    in_specs = [pl.BlockSpec((_BLK, d), lambda i: (i, 0))]
    for W in Ws:
        in_specs.append(pl.BlockSpec(W.shape, lambda i: (0, 0)))
    for b in bs:
        in_specs.append(pl.BlockSpec((1, b.shape[1]), lambda i: (0, 0)))
    out_specs = [pl.BlockSpec((_BLK, Ws[i].shape[0]), lambda i: (i, 0))
                 for i in range(nout)]
    return pl.pallas_call(
        functools.partial(_lin_body, nout),
        grid=grid,
        in_specs=in_specs,
        out_specs=out_specs,
        out_shape=out_shapes,
    )(x, *Ws, *bs)


def kernel(feat_word, feat_topic, effect, src_ww, dst_ww, ew_ww, src_wt, dst_wt, ew_wt, src_wd, dst_wd, ew_wd, src_td, dst_td, ew_td, src_tt, dst_tt, ew_tt, W_ww, b_ww, W_wt, b_wt, W_wd, b_wd, W_td, b_td, W_tt, b_tt, W_td_cau, b_td_cau, W_td_noi, b_td_noi, W_tt_cau, b_tt_cau, W_tt_noi, b_tt_noi, W_td_cau_trans, W_td_noi_trans, W_tt_cau_trans, W_tt_noi_trans):
    Nw, D = feat_word.shape
    Nt = feat_topic.shape[0]
    Nd = 50000

    Wh_ww, Wh_wt, Wh_wd = _linears(
        feat_word, [W_ww, W_wt, W_wd],
        [b_ww.reshape(1, -1), b_wt.reshape(1, -1), b_wd.reshape(1, -1)])
    Wh_td, Wh_tt = _linears(
        feat_topic, [W_td, W_tt],
        [b_td.reshape(1, -1), b_tt.reshape(1, -1)])

    def agg(Wh, src, dst, ew, nd):
        m = jnp.take(Wh, src, axis=0) * ew
        msum = jax.ops.segment_sum(m, dst, num_segments=nd)
        deg = jax.ops.segment_sum(jnp.ones((dst.shape[0], 1), jnp.float32),
                                  dst, num_segments=nd)
        return msum / jnp.maximum(deg, 1.0)

    h_word = agg(Wh_ww, src_ww, dst_ww, ew_ww, Nw)
    h_topic = (agg(Wh_wt, src_wt, dst_wt, ew_wt, Nt)
               + agg(Wh_tt, src_tt, dst_tt, ew_tt, Nt))
    h_doc = (agg(Wh_wd, src_wd, dst_wd, ew_wd, Nd)
             + agg(Wh_td, src_td, dst_td, ew_td, Nd))
    return (h_word, h_topic, h_doc)


# speedup vs baseline: 1.9807x; 1.8832x over previous
import dataclasses
import functools

import jax
import jax.numpy as jnp
from jax import lax
from jax.experimental import pallas as pl
from jax.experimental.pallas import tpu as pltpu
from jax.experimental.pallas import tpu_sc as plsc

NC, NS, L = 2, 16, 16
D = 128
_BLK = 1000


def _lin_body(nout, x_ref, *refs):
    x = x_ref[...]
    for i in range(nout):
        W = refs[i][...]
        b = refs[nout + i][...]
        refs[2 * nout + i][...] = lax.dot_general(
            x, W, (((1,), (1,)), ((), ())),
            preferred_element_type=jnp.float32,
            precision=lax.Precision.HIGHEST,
        ) + b


def _linears(x, Ws, bs):
    n, d = x.shape
    nout = len(Ws)
    grid = (n // _BLK,)
    in_specs = [pl.BlockSpec((_BLK, d), lambda i: (i, 0))]
    in_specs += [pl.BlockSpec(W.shape, lambda i: (0, 0)) for W in Ws]
    in_specs += [pl.BlockSpec((1, b.shape[1]), lambda i: (0, 0)) for b in bs]
    return pl.pallas_call(
        functools.partial(_lin_body, nout),
        grid=grid,
        in_specs=in_specs,
        out_specs=[pl.BlockSpec((_BLK, W.shape[0]), lambda i: (i, 0))
                   for W in Ws],
        out_shape=[jax.ShapeDtypeStruct((n, W.shape[0]), jnp.float32)
                   for W in Ws],
    )(x, *Ws, *bs)


WSZ = 512
_PLEVEL = 3
_PROBE = False
GB = 128
ACC_H = 6528
ACC_T = 2 * ACC_H


def _ecfg(E):
    b = 16 * -(-E // (16 * NS))
    nw = -(-b // WSZ)
    padlen = (NS - 1) * b + WSZ * nw
    return dict(E=E, b=b, nw=nw, padlen=padlen)


ECFG = {'ww': _ecfg(200000), 'wt': _ecfg(100000), 'wd': _ecfg(200000),
        'td': _ecfg(50000), 'tt': _ecfg(50000)}

PLANS = [
    dict(out=0, npass=2, rows=12800, groups=[('ww', 0)]),
    dict(out=1, npass=1, rows=5120, groups=[('wt', 0), ('tt', ACC_H)]),
    dict(out=2, npass=4, rows=6400, groups=[('wd', 0), ('td', ACC_H)]),
]
OUT_PAD = [4 * 12800, 2 * 5120, 8 * 6400]


def _agg_body(wh_ww, wh_wt, wh_wd, wh_td, wh_tt,
              s_ww, d_ww, w_ww, s_wt, d_wt, w_wt, s_wd, d_wd, w_wd,
              s_td, d_td, w_td, s_tt, d_tt, w_tt,
              hw, ht, hd,
              acc_sh, deg_sh):
    pl.run_scoped(
        functools.partial(
            _agg_scoped,
            (wh_ww, wh_wt, wh_wd, wh_td, wh_tt),
            (s_ww, d_ww, w_ww, s_wt, d_wt, w_wt, s_wd, d_wd, w_wd,
             s_td, d_td, w_td, s_tt, d_tt, w_tt),
            (hw, ht, hd), acc_sh, deg_sh),
        pltpu.VMEM((1280,), jnp.int32),
        pltpu.VMEM((1280,), jnp.float32),
        pltpu.VMEM((16, GB), jnp.int32),
        pltpu.VMEM((GB, D), jnp.float32),
        pltpu.VMEM((WSZ,), jnp.int32),
        pltpu.VMEM((WSZ,), jnp.int32),
        pltpu.VMEM((WSZ,), jnp.float32),
        pltpu.VMEM((16, D), jnp.float32),
        pltpu.VMEM((16,), jnp.float32),
        pltpu.VMEM((16,), jnp.float32),
        pltpu.VMEM((16,), jnp.float32),
        pltpu.VMEM((GB,), jnp.float32),
    )


def _agg_scoped(whs_t, edges_t, outs_t, acc_sh, deg_sh,
                csrc, cew, coffs, rows_buf, wsrc, wdst, wew,
                stage, degstage, recip1, recip2,
                ones_v):
    (wh_ww, wh_wt, wh_wd, wh_td, wh_tt) = whs_t
    (s_ww, d_ww, w_ww, s_wt, d_wt, w_wt, s_wd, d_wd, w_wd,
     s_td, d_td, w_td, s_tt, d_tt, w_tt) = edges_t
    (hw, ht, hd) = outs_t
    core = lax.axis_index("c")
    sid = lax.axis_index("s")
    whs = {'ww': wh_ww, 'wt': wh_wt, 'wd': wh_wd, 'td': wh_td, 'tt': wh_tt}
    edges = {'ww': (s_ww, d_ww, w_ww), 'wt': (s_wt, d_wt, w_wt),
             'wd': (s_wd, d_wd, w_wd), 'td': (s_td, d_td, w_td),
             'tt': (s_tt, d_tt, w_tt)}
    outs = [hw, ht, hd]
    i16 = lax.iota(jnp.int32, 16)

    @pl.loop(0, 8)
    def _(j):
        ones_v[pl.ds(j * 16, 16)] = jnp.ones((16,), jnp.float32)

    def do_batch(et_wh, bi, acc_sh, deg_sh):
        pltpu.sync_copy(et_wh.at[csrc.at[pl.ds(bi * GB, GB)]], rows_buf)

        @pl.loop(0, GB)
        def _(r):
            ewv = plsc.load_gather(
                cew, [jnp.broadcast_to(bi * GB + r, (16,)).astype(jnp.int32)])
            for j in range(8):
                sl = pl.ds(j * 16, 16)
                rows_buf[r, sl] = rows_buf[r, sl] * ewv

        pltpu.sync_copy(rows_buf, acc_sh.at[coffs.at[bi]], add=True)
        pltpu.sync_copy(ones_v, deg_sh.at[coffs.at[bi]], add=True)

    def process(et, base, rows, acc_off, acc_sh, deg_sh):
        if _PLEVEL == 0:
            return
        cfg = ECFG[et]
        wh = whs[et]
        src_h, dst_h, ew_h = edges[et]
        stripe0 = sid * cfg['b']
        stripe_end = jnp.minimum(stripe0 + cfg['b'], cfg['E'])
        dummy = acc_off + rows + sid * 8

        def win_body(w, n):
            wb = stripe0 + w * WSZ
            pltpu.sync_copy(src_h.at[pl.ds(wb, WSZ)], wsrc)
            pltpu.sync_copy(dst_h.at[pl.ds(wb, WSZ)], wdst)
            pltpu.sync_copy(ew_h.at[pl.ds(wb, WSZ)], wew)

            def grp_body(g, n):
                s = wsrc[pl.ds(g * 16, 16)]
                d = wdst[pl.ds(g * 16, 16)]
                w_ = wew[pl.ds(g * 16, 16)]
                ge = wb + g * 16 + i16
                mi = ((d >= base) & (d < base + rows) & (ge < stripe_end))
                mcnt = jnp.cumsum(mi.astype(jnp.int32))
                pos = n + mcnt - 1
                plsc.store_scatter(csrc, [pos], s, mask=mi)
                plsc.store_scatter(cew, [pos], w_, mask=mi)
                offs = d - base + acc_off
                plsc.store_scatter(
                    coffs,
                    [lax.shift_right_logical(pos, 7),
                     lax.bitwise_and(pos, 127)],
                    offs, mask=mi)
                return n + jnp.sum(mi.astype(jnp.int32))

            n = lax.fori_loop(0, WSZ // 16, grp_body, n)

            nfull = lax.shift_right_logical(n, 7)

            def fl_body(bi, carry):
                do_batch(wh, bi, acc_sh, deg_sh)
                return carry

            lax.fori_loop(0, nfull, fl_body, jnp.int32(0))

            r = lax.bitwise_and(n, 127)
            mvbase = nfull * GB

            def mv_body(k, carry):
                sl_from = pl.ds(mvbase + k * 16, 16)
                sl_to = pl.ds(k * 16, 16)
                csrc[sl_to] = csrc[sl_from]
                cew[sl_to] = cew[sl_from]
                idx = k * 16 + i16
                v = plsc.load_gather(
                    coffs, [jnp.broadcast_to(nfull, (16,)).astype(jnp.int32),
                            idx])
                plsc.store_scatter(
                    coffs, [jnp.zeros((16,), jnp.int32), idx], v)
                return carry

            lax.fori_loop(0, lax.shift_right_logical(r + 15, 4), mv_body,
                          jnp.int32(0))
            return r

        n = lax.fori_loop(0, cfg['nw'], win_body, jnp.int32(0))

        start = lax.bitwise_and(n, ~jnp.int32(15))

        def pad_body(k, kcarry):
            idx = start + k * 16 + i16
            pm = idx >= n
            plsc.store_scatter(csrc, [idx], jnp.zeros((16,), jnp.int32),
                               mask=pm)
            plsc.store_scatter(cew, [idx], jnp.zeros((16,), jnp.float32),
                               mask=pm)
            plsc.store_scatter(
                coffs,
                [lax.shift_right_logical(idx, 7),
                 lax.bitwise_and(idx, 127)],
                jnp.broadcast_to(dummy, (16,)).astype(jnp.int32), mask=pm)
            return kcarry

        lax.fori_loop(0, lax.shift_right_logical(GB - start, 4), pad_body,
                      jnp.int32(0))

        def last_body(bi, carry):
            do_batch(wh, bi, acc_sh, deg_sh)
            return carry

        lax.fori_loop(0, lax.shift_right_logical(n + 127, 7), last_body,
                      jnp.int32(0))

    def normalize_out(out_ref, base, rows, two, acc_sh, deg_sh):
        rows_n = rows // NS
        off0 = sid * rows_n

        def tile_body(t, carry):
            gb0 = off0 + t * 16
            pltpu.sync_copy(acc_sh.at[pl.ds(gb0, 16)], stage)
            pltpu.sync_copy(deg_sh.at[pl.ds(gb0, 16)], degstage)
            dv = degstage[...]
            recip1[...] = 1.0 / jnp.maximum(dv, 1.0)
            if two:
                pltpu.sync_copy(acc_sh.at[pl.ds(ACC_H + gb0, 16)],
                                rows_buf.at[pl.ds(0, 16)])
                pltpu.sync_copy(deg_sh.at[pl.ds(ACC_H + gb0, 16)], degstage)
                dv2 = degstage[...]
                recip2[...] = 1.0 / jnp.maximum(dv2, 1.0)

            @pl.loop(0, 16)
            def _(r):
                rs = jnp.broadcast_to(r, (16,)).astype(jnp.int32)
                g1 = plsc.load_gather(recip1, [rs])
                if two:
                    g2 = plsc.load_gather(recip2, [rs])
                for j in range(8):
                    sl = pl.ds(j * 16, 16)
                    v = stage[r, sl] * g1
                    if two:
                        v = v + rows_buf[r, sl] * g2
                    stage[r, sl] = v

            pltpu.sync_copy(stage, out_ref.at[pl.ds(base + gb0, 16)])
            return carry

        lax.fori_loop(0, rows_n // 16, tile_body, jnp.int32(0))

    def run_plan(plan, acc_sh, deg_sh):
        rows = plan['rows']
        out_ref = outs[plan['out']]

        def pass_body(p, pcarry):
            base = (core * plan['npass'] + p) * rows

            @pl.loop(0, 16)
            def _(r):
                for j in range(8):
                    stage[r, pl.ds(j * 16, 16)] = jnp.zeros((16,),
                                                            jnp.float32)

            zb = sid * (ACC_T // NS)

            def z_body(k, carry):
                pltpu.sync_copy(stage, acc_sh.at[pl.ds(zb + k * 16, 16)])
                return carry

            lax.fori_loop(0, ACC_T // NS // 16, z_body, jnp.int32(0))
            recip1[...] = jnp.zeros((16,), jnp.float32)

            def zd_body(k, carry):
                pltpu.sync_copy(recip1, deg_sh.at[pl.ds(zb + k * 16, 16)])
                return carry

            lax.fori_loop(0, ACC_T // NS // 16, zd_body, jnp.int32(0))
            plsc.subcore_barrier()
            for (et, acc_off) in plan['groups']:
                process(et, base, rows, acc_off, acc_sh, deg_sh)
            plsc.subcore_barrier()
            normalize_out(out_ref, base, rows, len(plan['groups']) == 2,
                          acc_sh, deg_sh)
            plsc.subcore_barrier()
            return pcarry

        lax.fori_loop(0, plan['npass'], pass_body, jnp.int32(0))

    for plan in PLANS:
        run_plan(plan, acc_sh, deg_sh)


def _aggregate(whs, edges):
    mesh = plsc.VectorSubcoreMesh(core_axis_name="c", subcore_axis_name="s")
    out_type = [jax.ShapeDtypeStruct((OUT_PAD[i], D), jnp.float32)
                for i in range(3)]
    scratch = [
        pltpu.VMEM_SHARED((ACC_T, D), jnp.float32),
        pltpu.VMEM_SHARED((ACC_T,), jnp.float32),
    ]
    cp = pltpu.CompilerParams()
    if "needs_layout_passes" in pltpu.CompilerParams.__dataclass_fields__:
        cp = dataclasses.replace(cp, needs_layout_passes=False)
    fn = pl.kernel(_agg_body, out_type=out_type, mesh=mesh,
                   scratch_types=scratch, compiler_params=cp)
    args = list(whs) + [a for e in edges for a in e]
    return fn(*args)


def kernel(feat_word, feat_topic, effect, src_ww, dst_ww, ew_ww, src_wt, dst_wt, ew_wt, src_wd, dst_wd, ew_wd, src_td, dst_td, ew_td, src_tt, dst_tt, ew_tt, W_ww, b_ww, W_wt, b_wt, W_wd, b_wd, W_td, b_td, W_tt, b_tt, W_td_cau, b_td_cau, W_td_noi, b_td_noi, W_tt_cau, b_tt_cau, W_tt_noi, b_tt_noi, W_td_cau_trans, W_td_noi_trans, W_tt_cau_trans, W_tt_noi_trans):
    Nw = feat_word.shape[0]
    Nt = feat_topic.shape[0]
    Nd = 50000

    Wh_ww, Wh_wt, Wh_wd = _linears(
        feat_word, [W_ww, W_wt, W_wd],
        [b_ww.reshape(1, -1), b_wt.reshape(1, -1), b_wd.reshape(1, -1)])
    Wh_td, Wh_tt = _linears(
        feat_topic, [W_td, W_tt],
        [b_td.reshape(1, -1), b_tt.reshape(1, -1)])

    def pad_edges(et, src, dst, ew):
        cfg = ECFG[et]
        p = cfg['padlen'] - cfg['E']
        src = jnp.pad(src, (0, p))
        dst = jnp.pad(dst, (0, p), constant_values=jnp.int32(2**30))
        ew = jnp.pad(ew.reshape(-1), (0, p))
        return (src, dst, ew)

    edges = [pad_edges('ww', src_ww, dst_ww, ew_ww),
             pad_edges('wt', src_wt, dst_wt, ew_wt),
             pad_edges('wd', src_wd, dst_wd, ew_wd),
             pad_edges('td', src_td, dst_td, ew_td),
             pad_edges('tt', src_tt, dst_tt, ew_tt)]

    hw, ht, hd = _aggregate([Wh_ww, Wh_wt, Wh_wd, Wh_td, Wh_tt], edges)
    return (hw[:Nw], ht[:Nt], hd[:Nd])


# speedup vs baseline: 2.4452x; 1.2345x over previous
import dataclasses
import functools

import jax
import jax.numpy as jnp
from jax import lax
from jax.experimental import pallas as pl
from jax.experimental.pallas import tpu as pltpu
from jax.experimental.pallas import tpu_sc as plsc

NC, NS, L = 2, 16, 16
D = 128
_BLK = 1000


def _lin_body(nout, x_ref, *refs):
    x = x_ref[...]
    for i in range(nout):
        W = refs[i][...]
        b = refs[nout + i][...]
        refs[2 * nout + i][...] = lax.dot_general(
            x, W, (((1,), (1,)), ((), ())),
            preferred_element_type=jnp.float32,
            precision=lax.Precision.HIGHEST,
        ) + b


def _linears(x, Ws, bs):
    n, d = x.shape
    nout = len(Ws)
    grid = (n // _BLK,)
    in_specs = [pl.BlockSpec((_BLK, d), lambda i: (i, 0))]
    in_specs += [pl.BlockSpec(W.shape, lambda i: (0, 0)) for W in Ws]
    in_specs += [pl.BlockSpec((1, b.shape[1]), lambda i: (0, 0)) for b in bs]
    return pl.pallas_call(
        functools.partial(_lin_body, nout),
        grid=grid,
        in_specs=in_specs,
        out_specs=[pl.BlockSpec((_BLK, W.shape[0]), lambda i: (i, 0))
                   for W in Ws],
        out_shape=[jax.ShapeDtypeStruct((n, W.shape[0]), jnp.float32)
                   for W in Ws],
    )(x, *Ws, *bs)


WSZ = 512
GB = 128
CAP = 768
NBMAX = 8
ACC_H = 6528
ACC_T = 2 * ACC_H
ZCH = ACC_T // NS


def _ecfg(E):
    b = WSZ * -(-E // (WSZ * NS))
    return dict(E=E, b=b, nw=b // WSZ, padlen=NS * b)


ECFG = {'ww': _ecfg(200000), 'wt': _ecfg(100000), 'wd': _ecfg(200000),
        'td': _ecfg(50000), 'tt': _ecfg(50000)}

PLANS = [
    dict(out=0, npass=2, rows=12800, groups=[('ww', 0)]),
    dict(out=1, npass=1, rows=5120, groups=[('wt', 0), ('tt', ACC_H)]),
    dict(out=2, npass=4, rows=6400, groups=[('wd', 0), ('td', ACC_H)]),
]
OUT_PAD = [4 * 12800, 2 * 5120, 8 * 6400]


def _agg_body(wh_ww, wh_wt, wh_wd, wh_td, wh_tt,
              e_ww, e_wt, e_wd, e_td, e_tt,
              hw, ht, hd,
              acc_sh, deg_sh, wsem0, wsem1, dsem, zsem, nsem):
    pl.run_scoped(
        functools.partial(
            _agg_scoped,
            (wh_ww, wh_wt, wh_wd, wh_td, wh_tt),
            (e_ww, e_wt, e_wd, e_td, e_tt),
            (hw, ht, hd), acc_sh, deg_sh,
            (wsem0, wsem1), dsem, zsem, nsem),
        pltpu.VMEM((CAP,), jnp.int32),
        pltpu.VMEM((CAP,), jnp.float32),
        pltpu.VMEM((NBMAX, GB), jnp.int32),
        pltpu.VMEM((GB, D), jnp.float32),
        pltpu.VMEM((2 * 3 * WSZ,), jnp.int32),
        pltpu.VMEM((64,), jnp.float32),
        pltpu.VMEM((64,), jnp.float32),
        pltpu.VMEM((64,), jnp.float32),
        pltpu.VMEM((64,), jnp.float32),
        pltpu.VMEM((GB,), jnp.float32),
    )


def _agg_scoped(whs_t, ecombs_t, outs_t, acc_sh, deg_sh, wsems, dsem,
                zsem, nsem,
                csrc, cew, coffs, rows_buf, wcomb,
                degstage, recip1, recip2, dzero, ones_v):
    whs = dict(zip(['ww', 'wt', 'wd', 'td', 'tt'], whs_t))
    ecombs = dict(zip(['ww', 'wt', 'wd', 'td', 'tt'], ecombs_t))
    outs = list(outs_t)
    core = lax.axis_index("c")
    sid = lax.axis_index("s")
    i16 = lax.iota(jnp.int32, 16)

    @pl.loop(0, GB // 16)
    def _(j):
        ones_v[pl.ds(j * 16, 16)] = jnp.ones((16,), jnp.float32)

    @pl.loop(0, 4)
    def _(j):
        dzero[pl.ds(j * 16, 16)] = jnp.zeros((16,), jnp.float32)

    def do_batch(wh, bi):
        pltpu.sync_copy(wh.at[csrc.at[pl.ds(bi * GB, GB)]], rows_buf)

        @pl.loop(0, GB)
        def _(r):
            ewv = plsc.load_gather(
                cew, [jnp.broadcast_to(bi * GB + r, (16,)).astype(jnp.int32)])
            for j in range(8):
                sl = pl.ds(j * 16, 16)
                rows_buf[r, sl] = rows_buf[r, sl] * ewv

        pltpu.sync_copy(rows_buf, acc_sh.at[coffs.at[bi]], add=True)
        pltpu.async_copy(ones_v, deg_sh.at[coffs.at[bi]], dsem, add=True)

    def process(et, base, rows, acc_off):
        cfg = ECFG[et]
        wh = whs[et]
        ecomb = ecombs[et]
        nw = cfg['nw']
        blk0 = sid * nw
        stripe_end = jnp.minimum((sid + 1) * cfg['b'], cfg['E'])
        dummy = acc_off + rows + sid * 8

        def w_issue(w, par):
            pltpu.async_copy(
                ecomb.at[pl.ds((blk0 + w) * 3 * WSZ, 3 * WSZ)],
                wcomb.at[pl.ds(par * 3 * WSZ, 3 * WSZ)], wsems[par])

        def w_wait(w, par):
            pltpu.make_async_copy(
                ecomb.at[pl.ds((blk0 + w) * 3 * WSZ, 3 * WSZ)],
                wcomb.at[pl.ds(par * 3 * WSZ, 3 * WSZ)], wsems[par]).wait()

        def scan_flush(w, par, n):
            w_wait(w, par)
            w_issue(w + 1, par ^ 1)
            pbase = par * 3 * WSZ
            wb = sid * cfg['b'] + w * WSZ

            def grp_body(g, n):
                s = wcomb[pl.ds(pbase + g * 16, 16)]
                d = wcomb[pl.ds(pbase + WSZ + g * 16, 16)]
                w_ = plsc.bitcast(wcomb[pl.ds(pbase + 2 * WSZ + g * 16, 16)],
                                  jnp.float32)
                ge = wb + g * 16 + i16
                mi = ((d >= base) & (d < base + rows) & (ge < stripe_end))
                mcnt = jnp.cumsum(mi.astype(jnp.int32))
                pos = n + mcnt - 1
                plsc.store_scatter(csrc, [pos], s, mask=mi)
                plsc.store_scatter(cew, [pos], w_, mask=mi)
                offs = d - base + acc_off
                plsc.store_scatter(
                    coffs,
                    [lax.shift_right_logical(pos, 7),
                     lax.bitwise_and(pos, 127)],
                    offs, mask=mi)
                return n + jnp.sum(mi.astype(jnp.int32))

            n = lax.fori_loop(0, WSZ // 16, grp_body, n)

            nfull = lax.shift_right_logical(n, 7)

            def fl_body(bi, carry):
                do_batch(wh, bi)
                return carry

            lax.fori_loop(0, nfull, fl_body, jnp.int32(0))

            def dr_body(k, carry):
                pltpu.make_async_copy(ones_v, deg_sh.at[coffs.at[0]],
                                      dsem).wait()
                return carry

            lax.fori_loop(0, nfull, dr_body, jnp.int32(0))

            r = lax.bitwise_and(n, 127)
            mvbase = nfull * GB

            def mv_body(k, carry2):
                sl_from = pl.ds(mvbase + k * 16, 16)
                sl_to = pl.ds(k * 16, 16)
                csrc[sl_to] = csrc[sl_from]
                cew[sl_to] = cew[sl_from]
                idx = k * 16 + i16
                v = plsc.load_gather(
                    coffs, [jnp.broadcast_to(nfull, (16,)).astype(jnp.int32),
                            idx])
                plsc.store_scatter(
                    coffs, [jnp.zeros((16,), jnp.int32), idx], v)
                return carry2

            lax.fori_loop(0, lax.shift_right_logical(r + 15, 4), mv_body,
                          jnp.int32(0))
            return r

        w_issue(0, 0)

        def pair_body(i, n):
            n = scan_flush(2 * i, 0, n)
            n = scan_flush(2 * i + 1, 1, n)
            return n

        n = lax.fori_loop(0, nw // 2, pair_body, jnp.int32(0))
        if nw % 2:
            n = scan_flush(nw - 1, 0, n)
        w_wait(nw, nw % 2)

        start = lax.bitwise_and(n, ~jnp.int32(15))

        def pad_body(k, kcarry):
            idx = start + k * 16 + i16
            pm = idx >= n
            plsc.store_scatter(csrc, [idx], jnp.zeros((16,), jnp.int32),
                               mask=pm)
            plsc.store_scatter(cew, [idx], jnp.zeros((16,), jnp.float32),
                               mask=pm)
            plsc.store_scatter(
                coffs,
                [lax.shift_right_logical(idx, 7),
                 lax.bitwise_and(idx, 127)],
                jnp.broadcast_to(dummy, (16,)).astype(jnp.int32), mask=pm)
            return kcarry

        lax.fori_loop(0, lax.shift_right_logical(GB - start, 4), pad_body,
                      jnp.int32(0))

        nlast = lax.shift_right_logical(n + (GB - 1), 7)

        def last_body(bi, carry):
            do_batch(wh, bi)
            return carry

        lax.fori_loop(0, nlast, last_body, jnp.int32(0))

        def drl_body(k, carry):
            pltpu.make_async_copy(ones_v, deg_sh.at[coffs.at[0]],
                                  dsem).wait()
            return carry

        lax.fori_loop(0, nlast, drl_body, jnp.int32(0))

    def norm_tile(out_ref, base, two, t0, sz):
        stg = rows_buf.at[pl.ds(0, sz)]
        stg2 = rows_buf.at[pl.ds(64, sz)]
        pltpu.async_copy(acc_sh.at[pl.ds(t0, sz)], stg, nsem)
        pltpu.async_copy(deg_sh.at[pl.ds(t0, sz)],
                         degstage.at[pl.ds(0, sz)], nsem)
        if two:
            pltpu.async_copy(acc_sh.at[pl.ds(ACC_H + t0, sz)], stg2, nsem)
        pltpu.make_async_copy(acc_sh.at[pl.ds(t0, sz)], stg, nsem).wait()
        pltpu.make_async_copy(deg_sh.at[pl.ds(t0, sz)],
                              degstage.at[pl.ds(0, sz)], nsem).wait()
        if two:
            pltpu.make_async_copy(acc_sh.at[pl.ds(ACC_H + t0, sz)], stg2,
                                  nsem).wait()

        @pl.loop(0, sz // 16)
        def _(k):
            dv = degstage[pl.ds(k * 16, 16)]
            recip1[pl.ds(k * 16, 16)] = 1.0 / jnp.maximum(dv, 1.0)

        if two:
            pltpu.sync_copy(deg_sh.at[pl.ds(ACC_H + t0, sz)],
                            degstage.at[pl.ds(0, sz)])

            @pl.loop(0, sz // 16)
            def _(k):
                dv = degstage[pl.ds(k * 16, 16)]
                recip2[pl.ds(k * 16, 16)] = 1.0 / jnp.maximum(dv, 1.0)

        @pl.loop(0, sz)
        def _(r):
            rs = jnp.broadcast_to(r, (16,)).astype(jnp.int32)
            g1 = plsc.load_gather(recip1, [rs])
            if two:
                g2 = plsc.load_gather(recip2, [rs])
            for j in range(8):
                sl = pl.ds(j * 16, 16)
                v = rows_buf[r, sl] * g1
                if two:
                    v = v + rows_buf[64 + r, sl] * g2
                rows_buf[r, sl] = v

        pltpu.sync_copy(stg, out_ref.at[pl.ds(base + t0, sz)])

    def normalize_out(out_ref, base, rows, two):
        rows_n = rows // NS
        off0 = sid * rows_n

        def tile_body(t, carry):
            norm_tile(out_ref, base, two, off0 + t * 64, 64)
            return carry

        lax.fori_loop(0, rows_n // 64, tile_body, jnp.int32(0))
        if rows_n % 64:
            norm_tile(out_ref, base, two, off0 + (rows_n // 64) * 64,
                      rows_n % 64)

    def run_plan(plan):
        rows = plan['rows']
        out_ref = outs[plan['out']]

        def pass_body(p, pcarry):
            base = (core * plan['npass'] + p) * rows

            @pl.loop(0, GB)
            def _(r):
                for j in range(8):
                    rows_buf[r, pl.ds(j * 16, 16)] = jnp.zeros(
                        (16,), jnp.float32)

            zb = sid * ZCH
            nz = ZCH // 64
            zr = ZCH - nz * 64

            def z_issue(k, carry):
                pltpu.async_copy(rows_buf.at[pl.ds(0, 64)],
                                 acc_sh.at[pl.ds(zb + k * 64, 64)], zsem)
                pltpu.async_copy(dzero, deg_sh.at[pl.ds(zb + k * 64, 64)],
                                 zsem)
                return carry

            lax.fori_loop(0, nz, z_issue, jnp.int32(0))
            pltpu.async_copy(rows_buf.at[pl.ds(0, zr)],
                             acc_sh.at[pl.ds(zb + nz * 64, zr)], zsem)
            pltpu.async_copy(dzero.at[pl.ds(0, zr - 16)],
                             deg_sh.at[pl.ds(zb + nz * 64, zr - 16)], zsem)
            pltpu.async_copy(dzero.at[pl.ds(0, 16)],
                             deg_sh.at[pl.ds(zb + nz * 64 + zr - 16, 16)],
                             zsem)

            def z_drain(k, carry):
                pltpu.make_async_copy(
                    rows_buf.at[pl.ds(0, 64)],
                    acc_sh.at[pl.ds(zb + k * 64, 64)], zsem).wait()
                pltpu.make_async_copy(
                    dzero, deg_sh.at[pl.ds(zb + k * 64, 64)], zsem).wait()
                return carry

            lax.fori_loop(0, nz, z_drain, jnp.int32(0))
            pltpu.make_async_copy(rows_buf.at[pl.ds(0, zr)],
                                  acc_sh.at[pl.ds(zb + nz * 64, zr)],
                                  zsem).wait()
            pltpu.make_async_copy(dzero.at[pl.ds(0, zr - 16)],
                                  deg_sh.at[pl.ds(zb + nz * 64, zr - 16)],
                                  zsem).wait()
            pltpu.make_async_copy(
                dzero.at[pl.ds(0, 16)],
                deg_sh.at[pl.ds(zb + nz * 64 + zr - 16, 16)], zsem).wait()

            plsc.subcore_barrier()
            for (et, acc_off) in plan['groups']:
                process(et, base, rows, acc_off)
            plsc.subcore_barrier()
            normalize_out(out_ref, base, rows, len(plan['groups']) == 2)
            plsc.subcore_barrier()
            return pcarry

        lax.fori_loop(0, plan['npass'], pass_body, jnp.int32(0))

    for plan in PLANS:
        run_plan(plan)


def _aggregate(whs, ecombs):
    mesh = plsc.VectorSubcoreMesh(core_axis_name="c", subcore_axis_name="s")
    out_type = [jax.ShapeDtypeStruct((OUT_PAD[i], D), jnp.float32)
                for i in range(3)]
    scratch = [
        pltpu.VMEM_SHARED((ACC_T, D), jnp.float32),
        pltpu.VMEM_SHARED((ACC_T,), jnp.float32),
        pltpu.SemaphoreType.DMA,
        pltpu.SemaphoreType.DMA,
        pltpu.SemaphoreType.DMA,
        pltpu.SemaphoreType.DMA,
        pltpu.SemaphoreType.DMA,
    ]
    cp = pltpu.CompilerParams()
    if "needs_layout_passes" in pltpu.CompilerParams.__dataclass_fields__:
        cp = dataclasses.replace(cp, needs_layout_passes=False)
    fn = pl.kernel(_agg_body, out_type=out_type, mesh=mesh,
                   scratch_types=scratch, compiler_params=cp)
    return fn(*whs, *ecombs)


def kernel(feat_word, feat_topic, effect, src_ww, dst_ww, ew_ww, src_wt, dst_wt, ew_wt, src_wd, dst_wd, ew_wd, src_td, dst_td, ew_td, src_tt, dst_tt, ew_tt, W_ww, b_ww, W_wt, b_wt, W_wd, b_wd, W_td, b_td, W_tt, b_tt, W_td_cau, b_td_cau, W_td_noi, b_td_noi, W_tt_cau, b_tt_cau, W_tt_noi, b_tt_noi, W_td_cau_trans, W_td_noi_trans, W_tt_cau_trans, W_tt_noi_trans):
    Nw = feat_word.shape[0]
    Nt = feat_topic.shape[0]
    Nd = 50000

    Wh_ww, Wh_wt, Wh_wd = _linears(
        feat_word, [W_ww, W_wt, W_wd],
        [b_ww.reshape(1, -1), b_wt.reshape(1, -1), b_wd.reshape(1, -1)])
    Wh_td, Wh_tt = _linears(
        feat_topic, [W_td, W_tt],
        [b_td.reshape(1, -1), b_tt.reshape(1, -1)])

    def interleave(et, src, dst, ew):
        cfg = ECFG[et]
        p = cfg['padlen'] - cfg['E']
        srcw = jnp.pad(src, (0, p)).reshape(-1, WSZ)
        dstw = jnp.pad(dst, (0, p),
                       constant_values=jnp.int32(2**30)).reshape(-1, WSZ)
        ewb = lax.bitcast_convert_type(
            jnp.pad(ew.reshape(-1), (0, p)), jnp.int32).reshape(-1, WSZ)
        flat = jnp.stack([srcw, dstw, ewb], axis=1).reshape(-1)
        return jnp.pad(flat, (0, 3 * WSZ))

    ecombs = [interleave('ww', src_ww, dst_ww, ew_ww),
              interleave('wt', src_wt, dst_wt, ew_wt),
              interleave('wd', src_wd, dst_wd, ew_wd),
              interleave('td', src_td, dst_td, ew_td),
              interleave('tt', src_tt, dst_tt, ew_tt)]

    hw, ht, hd = _aggregate([Wh_ww, Wh_wt, Wh_wd, Wh_td, Wh_tt], ecombs)
    return (hw[:Nw], ht[:Nt], hd[:Nd])


# speedup vs baseline: 3.1303x; 1.2802x over previous
import dataclasses
import functools

import jax
import jax.numpy as jnp
from jax import lax
from jax.experimental import pallas as pl
from jax.experimental.pallas import tpu as pltpu
from jax.experimental.pallas import tpu_sc as plsc

NC, NS, L = 2, 16, 16
D = 128
_BLK = 1000


def _lin_body(nout, x_ref, *refs):
    x = x_ref[...]
    for i in range(nout):
        W = refs[i][...]
        b = refs[nout + i][...]
        refs[2 * nout + i][...] = lax.dot_general(
            x, W, (((1,), (1,)), ((), ())),
            preferred_element_type=jnp.float32,
            precision=lax.Precision.HIGHEST,
        ) + b


def _linears(x, Ws, bs):
    n, d = x.shape
    nout = len(Ws)
    grid = (n // _BLK,)
    in_specs = [pl.BlockSpec((_BLK, d), lambda i: (i, 0))]
    in_specs += [pl.BlockSpec(W.shape, lambda i: (0, 0)) for W in Ws]
    in_specs += [pl.BlockSpec((1, b.shape[1]), lambda i: (0, 0)) for b in bs]
    return pl.pallas_call(
        functools.partial(_lin_body, nout),
        grid=grid,
        in_specs=in_specs,
        out_specs=[pl.BlockSpec((_BLK, W.shape[0]), lambda i: (i, 0))
                   for W in Ws],
        out_shape=[jax.ShapeDtypeStruct((n, W.shape[0]), jnp.float32)
                   for W in Ws],
    )(x, *Ws, *bs)


WSZ = 512
GB = 64
CAP = 768
NBMAX = 16
ACC_H = 6528
ACC_T = 2 * ACC_H
ZCH = ACC_T // NS


def _ecfg(E):
    b = WSZ * -(-E // (WSZ * NS))
    return dict(E=E, b=b, nw=b // WSZ, padlen=NS * b)


ECFG = {'ww': _ecfg(200000), 'wt': _ecfg(100000), 'wd': _ecfg(200000),
        'td': _ecfg(50000), 'tt': _ecfg(50000)}

PLANS = [
    dict(out=0, npass=2, rows=12800, groups=[('ww', 0)]),
    dict(out=1, npass=1, rows=5120, groups=[('wt', 0), ('tt', ACC_H)]),
    dict(out=2, npass=4, rows=6400, groups=[('wd', 0), ('td', ACC_H)]),
]
OUT_PAD = [4 * 12800, 2 * 5120, 8 * 6400]


def _agg_body(wh_ww, wh_wt, wh_wd, wh_td, wh_tt,
              e_ww, e_wt, e_wd, e_td, e_tt,
              hw, ht, hd,
              acc_sh, deg_sh, wsem0, wsem1, dsem, zsem, nsem,
              gsem0, gsem1, ssem0, ssem1):
    pl.run_scoped(
        functools.partial(
            _agg_scoped,
            (wh_ww, wh_wt, wh_wd, wh_td, wh_tt),
            (e_ww, e_wt, e_wd, e_td, e_tt),
            (hw, ht, hd), acc_sh, deg_sh,
            (wsem0, wsem1), dsem, zsem, nsem,
            (gsem0, gsem1), (ssem0, ssem1)),
        pltpu.VMEM((CAP,), jnp.int32),
        pltpu.VMEM((CAP,), jnp.float32),
        pltpu.VMEM((NBMAX, GB), jnp.int32),
        pltpu.VMEM((2 * GB, D), jnp.float32),
        pltpu.VMEM((2 * 3 * WSZ,), jnp.int32),
        pltpu.VMEM((64,), jnp.float32),
        pltpu.VMEM((64,), jnp.float32),
        pltpu.VMEM((64,), jnp.float32),
        pltpu.VMEM((64,), jnp.float32),
        pltpu.VMEM((GB,), jnp.float32),
    )


def _agg_scoped(whs_t, ecombs_t, outs_t, acc_sh, deg_sh, wsems, dsem,
                zsem, nsem, gsems, ssems,
                csrc, cew, coffs, rows_buf, wcomb,
                degstage, recip1, recip2, dzero, ones_v):
    whs = dict(zip(['ww', 'wt', 'wd', 'td', 'tt'], whs_t))
    ecombs = dict(zip(['ww', 'wt', 'wd', 'td', 'tt'], ecombs_t))
    outs = list(outs_t)
    core = lax.axis_index("c")
    sid = lax.axis_index("s")
    i16 = lax.iota(jnp.int32, 16)

    @pl.loop(0, GB // 16)
    def _(j):
        ones_v[pl.ds(j * 16, 16)] = jnp.ones((16,), jnp.float32)

    @pl.loop(0, 4)
    def _(j):
        dzero[pl.ds(j * 16, 16)] = jnp.zeros((16,), jnp.float32)

    def g_issue(wh, bi, par):
        pltpu.async_copy(wh.at[csrc.at[pl.ds(bi * GB, GB)]],
                         rows_buf.at[pl.ds(par * GB, GB)], gsems[par])

    def g_wait(wh, bi, par):
        pltpu.make_async_copy(wh.at[csrc.at[pl.ds(bi * GB, GB)]],
                              rows_buf.at[pl.ds(par * GB, GB)],
                              gsems[par]).wait()

    def s_issue(bi, par):
        pltpu.async_copy(rows_buf.at[pl.ds(par * GB, GB)],
                         acc_sh.at[coffs.at[bi]], ssems[par], add=True)
        pltpu.async_copy(ones_v, deg_sh.at[coffs.at[bi]], dsem, add=True)

    def s_wait(par):
        pltpu.make_async_copy(rows_buf.at[pl.ds(par * GB, GB)],
                              acc_sh.at[coffs.at[0]], ssems[par]).wait()

    def scale(bi, par):
        @pl.loop(0, GB)
        def _(r):
            ewv = plsc.load_gather(
                cew, [jnp.broadcast_to(bi * GB + r, (16,)).astype(jnp.int32)])
            for j in range(8):
                sl = pl.ds(j * 16, 16)
                rows_buf[par * GB + r, sl] = rows_buf[par * GB + r, sl] * ewv

    def flush(wh, nfull):
        @pl.when(nfull > 0)
        def _():
            g_issue(wh, 0, 0)

        def pair(i, carry):
            for par in (0, 1):
                b = 2 * i + par

                @pl.when(b < nfull)
                def _():
                    g_wait(wh, b, par)

                    @pl.when(b + 1 < nfull)
                    def _():
                        @pl.when(b >= 1)
                        def _():
                            s_wait(par ^ 1)

                        g_issue(wh, b + 1, par ^ 1)

                    scale(b, par)
                    s_issue(b, par)
            return carry

        lax.fori_loop(0, lax.shift_right_logical(nfull + 1, 1), pair,
                      jnp.int32(0))

        @pl.when(nfull >= 2)
        def _():
            s_wait(0)
            s_wait(1)

        @pl.when(nfull == 1)
        def _():
            s_wait(0)

        def drd(k, carry):
            pltpu.make_async_copy(ones_v, deg_sh.at[coffs.at[0]],
                                  dsem).wait()
            return carry

        lax.fori_loop(0, nfull, drd, jnp.int32(0))

    def process(et, base, rows, acc_off):
        cfg = ECFG[et]
        wh = whs[et]
        ecomb = ecombs[et]
        nw = cfg['nw']
        blk0 = sid * nw
        stripe_end = jnp.minimum((sid + 1) * cfg['b'], cfg['E'])
        dummy = acc_off + rows + sid * 8

        def w_issue(w, par):
            pltpu.async_copy(
                ecomb.at[pl.ds((blk0 + w) * 3 * WSZ, 3 * WSZ)],
                wcomb.at[pl.ds(par * 3 * WSZ, 3 * WSZ)], wsems[par])

        def w_wait(w, par):
            pltpu.make_async_copy(
                ecomb.at[pl.ds((blk0 + w) * 3 * WSZ, 3 * WSZ)],
                wcomb.at[pl.ds(par * 3 * WSZ, 3 * WSZ)], wsems[par]).wait()

        def scan_flush(w, par, n):
            w_wait(w, par)
            w_issue(w + 1, par ^ 1)
            pbase = par * 3 * WSZ
            wb = sid * cfg['b'] + w * WSZ

            def grp_body(g, n):
                s = wcomb[pl.ds(pbase + g * 16, 16)]
                d = wcomb[pl.ds(pbase + WSZ + g * 16, 16)]
                w_ = plsc.bitcast(wcomb[pl.ds(pbase + 2 * WSZ + g * 16, 16)],
                                  jnp.float32)
                ge = wb + g * 16 + i16
                mi = ((d >= base) & (d < base + rows) & (ge < stripe_end))
                mcnt = jnp.cumsum(mi.astype(jnp.int32))
                pos = n + mcnt - 1
                plsc.store_scatter(csrc, [pos], s, mask=mi)
                plsc.store_scatter(cew, [pos], w_, mask=mi)
                offs = d - base + acc_off
                plsc.store_scatter(
                    coffs,
                    [lax.shift_right_logical(pos, 6),
                     lax.bitwise_and(pos, 63)],
                    offs, mask=mi)
                return n + jnp.sum(mi.astype(jnp.int32))

            n = lax.fori_loop(0, WSZ // 16, grp_body, n)

            nfull = lax.shift_right_logical(n, 6)
            flush(wh, nfull)

            r = lax.bitwise_and(n, 63)
            mvbase = nfull * GB

            def mv_body(k, carry2):
                sl_from = pl.ds(mvbase + k * 16, 16)
                sl_to = pl.ds(k * 16, 16)
                csrc[sl_to] = csrc[sl_from]
                cew[sl_to] = cew[sl_from]
                idx = k * 16 + i16
                v = plsc.load_gather(
                    coffs, [jnp.broadcast_to(nfull, (16,)).astype(jnp.int32),
                            idx])
                plsc.store_scatter(
                    coffs, [jnp.zeros((16,), jnp.int32), idx], v)
                return carry2

            lax.fori_loop(0, lax.shift_right_logical(r + 15, 4), mv_body,
                          jnp.int32(0))
            return r

        w_issue(0, 0)

        def pair_body(i, n):
            n = scan_flush(2 * i, 0, n)
            n = scan_flush(2 * i + 1, 1, n)
            return n

        n = lax.fori_loop(0, nw // 2, pair_body, jnp.int32(0))
        if nw % 2:
            n = scan_flush(nw - 1, 0, n)
        w_wait(nw, nw % 2)

        start = lax.bitwise_and(n, ~jnp.int32(15))

        def pad_body(k, kcarry):
            idx = start + k * 16 + i16
            pm = idx >= n
            plsc.store_scatter(csrc, [idx], jnp.zeros((16,), jnp.int32),
                               mask=pm)
            plsc.store_scatter(cew, [idx], jnp.zeros((16,), jnp.float32),
                               mask=pm)
            plsc.store_scatter(
                coffs,
                [lax.shift_right_logical(idx, 7),
                 lax.bitwise_and(idx, 127)],
                jnp.broadcast_to(dummy, (16,)).astype(jnp.int32), mask=pm)
            return kcarry

        lax.fori_loop(0, lax.shift_right_logical(GB - start, 4), pad_body,
                      jnp.int32(0))

        nlast = lax.shift_right_logical(n + (GB - 1), 6)
        flush(wh, nlast)

    def norm_tile(out_ref, base, two, t0, sz):
        stg = rows_buf.at[pl.ds(0, sz)]
        stg2 = rows_buf.at[pl.ds(64, sz)]
        pltpu.async_copy(acc_sh.at[pl.ds(t0, sz)], stg, nsem)
        pltpu.async_copy(deg_sh.at[pl.ds(t0, sz)],
                         degstage.at[pl.ds(0, sz)], nsem)
        if two:
            pltpu.async_copy(acc_sh.at[pl.ds(ACC_H + t0, sz)], stg2, nsem)
        pltpu.make_async_copy(acc_sh.at[pl.ds(t0, sz)], stg, nsem).wait()
        pltpu.make_async_copy(deg_sh.at[pl.ds(t0, sz)],
                              degstage.at[pl.ds(0, sz)], nsem).wait()
        if two:
            pltpu.make_async_copy(acc_sh.at[pl.ds(ACC_H + t0, sz)], stg2,
                                  nsem).wait()

        @pl.loop(0, sz // 16)
        def _(k):
            dv = degstage[pl.ds(k * 16, 16)]
            recip1[pl.ds(k * 16, 16)] = 1.0 / jnp.maximum(dv, 1.0)

        if two:
            pltpu.sync_copy(deg_sh.at[pl.ds(ACC_H + t0, sz)],
                            degstage.at[pl.ds(0, sz)])

            @pl.loop(0, sz // 16)
            def _(k):
                dv = degstage[pl.ds(k * 16, 16)]
                recip2[pl.ds(k * 16, 16)] = 1.0 / jnp.maximum(dv, 1.0)

        @pl.loop(0, sz)
        def _(r):
            rs = jnp.broadcast_to(r, (16,)).astype(jnp.int32)
            g1 = plsc.load_gather(recip1, [rs])
            if two:
                g2 = plsc.load_gather(recip2, [rs])
            for j in range(8):
                sl = pl.ds(j * 16, 16)
                v = rows_buf[r, sl] * g1
                if two:
                    v = v + rows_buf[64 + r, sl] * g2
                rows_buf[r, sl] = v

        pltpu.sync_copy(stg, out_ref.at[pl.ds(base + t0, sz)])

    def normalize_out(out_ref, base, rows, two):
        rows_n = rows // NS
        off0 = sid * rows_n

        def tile_body(t, carry):
            norm_tile(out_ref, base, two, off0 + t * 64, 64)
            return carry

        lax.fori_loop(0, rows_n // 64, tile_body, jnp.int32(0))
        if rows_n % 64:
            norm_tile(out_ref, base, two, off0 + (rows_n // 64) * 64,
                      rows_n % 64)

    def run_plan(plan):
        rows = plan['rows']
        out_ref = outs[plan['out']]

        def pass_body(p, pcarry):
            base = (core * plan['npass'] + p) * rows

            @pl.loop(0, GB)
            def _(r):
                for j in range(8):
                    rows_buf[r, pl.ds(j * 16, 16)] = jnp.zeros(
                        (16,), jnp.float32)

            zb = sid * ZCH
            nz = ZCH // 64
            zr = ZCH - nz * 64

            def z_issue(k, carry):
                pltpu.async_copy(rows_buf.at[pl.ds(0, 64)],
                                 acc_sh.at[pl.ds(zb + k * 64, 64)], zsem)
                pltpu.async_copy(dzero, deg_sh.at[pl.ds(zb + k * 64, 64)],
                                 zsem)
                return carry

            lax.fori_loop(0, nz, z_issue, jnp.int32(0))
            pltpu.async_copy(rows_buf.at[pl.ds(0, zr)],
                             acc_sh.at[pl.ds(zb + nz * 64, zr)], zsem)
            pltpu.async_copy(dzero.at[pl.ds(0, zr - 16)],
                             deg_sh.at[pl.ds(zb + nz * 64, zr - 16)], zsem)
            pltpu.async_copy(dzero.at[pl.ds(0, 16)],
                             deg_sh.at[pl.ds(zb + nz * 64 + zr - 16, 16)],
                             zsem)

            def z_drain(k, carry):
                pltpu.make_async_copy(
                    rows_buf.at[pl.ds(0, 64)],
                    acc_sh.at[pl.ds(zb + k * 64, 64)], zsem).wait()
                pltpu.make_async_copy(
                    dzero, deg_sh.at[pl.ds(zb + k * 64, 64)], zsem).wait()
                return carry

            lax.fori_loop(0, nz, z_drain, jnp.int32(0))
            pltpu.make_async_copy(rows_buf.at[pl.ds(0, zr)],
                                  acc_sh.at[pl.ds(zb + nz * 64, zr)],
                                  zsem).wait()
            pltpu.make_async_copy(dzero.at[pl.ds(0, zr - 16)],
                                  deg_sh.at[pl.ds(zb + nz * 64, zr - 16)],
                                  zsem).wait()
            pltpu.make_async_copy(
                dzero.at[pl.ds(0, 16)],
                deg_sh.at[pl.ds(zb + nz * 64 + zr - 16, 16)], zsem).wait()

            plsc.subcore_barrier()
            for (et, acc_off) in plan['groups']:
                process(et, base, rows, acc_off)
            plsc.subcore_barrier()
            normalize_out(out_ref, base, rows, len(plan['groups']) == 2)
            plsc.subcore_barrier()
            return pcarry

        lax.fori_loop(0, plan['npass'], pass_body, jnp.int32(0))

    for plan in PLANS:
        run_plan(plan)


def _aggregate(whs, ecombs):
    mesh = plsc.VectorSubcoreMesh(core_axis_name="c", subcore_axis_name="s")
    out_type = [jax.ShapeDtypeStruct((OUT_PAD[i], D), jnp.float32)
                for i in range(3)]
    scratch = [
        pltpu.VMEM_SHARED((ACC_T, D), jnp.float32),
        pltpu.VMEM_SHARED((ACC_T,), jnp.float32),
        pltpu.SemaphoreType.DMA,
        pltpu.SemaphoreType.DMA,
        pltpu.SemaphoreType.DMA,
        pltpu.SemaphoreType.DMA,
        pltpu.SemaphoreType.DMA,
        pltpu.SemaphoreType.DMA,
        pltpu.SemaphoreType.DMA,
        pltpu.SemaphoreType.DMA,
        pltpu.SemaphoreType.DMA,
    ]
    cp = pltpu.CompilerParams()
    if "needs_layout_passes" in pltpu.CompilerParams.__dataclass_fields__:
        cp = dataclasses.replace(cp, needs_layout_passes=False)
    fn = pl.kernel(_agg_body, out_type=out_type, mesh=mesh,
                   scratch_types=scratch, compiler_params=cp)
    return fn(*whs, *ecombs)


def kernel(feat_word, feat_topic, effect, src_ww, dst_ww, ew_ww, src_wt, dst_wt, ew_wt, src_wd, dst_wd, ew_wd, src_td, dst_td, ew_td, src_tt, dst_tt, ew_tt, W_ww, b_ww, W_wt, b_wt, W_wd, b_wd, W_td, b_td, W_tt, b_tt, W_td_cau, b_td_cau, W_td_noi, b_td_noi, W_tt_cau, b_tt_cau, W_tt_noi, b_tt_noi, W_td_cau_trans, W_td_noi_trans, W_tt_cau_trans, W_tt_noi_trans):
    Nw = feat_word.shape[0]
    Nt = feat_topic.shape[0]
    Nd = 50000

    Wh_ww, Wh_wt, Wh_wd = _linears(
        feat_word, [W_ww, W_wt, W_wd],
        [b_ww.reshape(1, -1), b_wt.reshape(1, -1), b_wd.reshape(1, -1)])
    Wh_td, Wh_tt = _linears(
        feat_topic, [W_td, W_tt],
        [b_td.reshape(1, -1), b_tt.reshape(1, -1)])

    def interleave(et, src, dst, ew):
        cfg = ECFG[et]
        p = cfg['padlen'] - cfg['E']
        srcw = jnp.pad(src, (0, p)).reshape(-1, WSZ)
        dstw = jnp.pad(dst, (0, p),
                       constant_values=jnp.int32(2**30)).reshape(-1, WSZ)
        ewb = lax.bitcast_convert_type(
            jnp.pad(ew.reshape(-1), (0, p)), jnp.int32).reshape(-1, WSZ)
        flat = jnp.stack([srcw, dstw, ewb], axis=1).reshape(-1)
        return jnp.pad(flat, (0, 3 * WSZ))

    ecombs = [interleave('ww', src_ww, dst_ww, ew_ww),
              interleave('wt', src_wt, dst_wt, ew_wt),
              interleave('wd', src_wd, dst_wd, ew_wd),
              interleave('td', src_td, dst_td, ew_td),
              interleave('tt', src_tt, dst_tt, ew_tt)]

    hw, ht, hd = _aggregate([Wh_ww, Wh_wt, Wh_wd, Wh_td, Wh_tt], ecombs)
    return (hw[:Nw], ht[:Nt], hd[:Nd])


# speedup vs baseline: 3.1358x; 1.0018x over previous
import dataclasses
import functools

import jax
import jax.numpy as jnp
from jax import lax
from jax.experimental import pallas as pl
from jax.experimental.pallas import tpu as pltpu
from jax.experimental.pallas import tpu_sc as plsc

NC, NS, L = 2, 16, 16
D = 128
_BLK = 1000


def _lin_body(nout, x_ref, *refs):
    x = x_ref[...]
    for i in range(nout):
        W = refs[i][...]
        b = refs[nout + i][...]
        refs[2 * nout + i][...] = lax.dot_general(
            x, W, (((1,), (1,)), ((), ())),
            preferred_element_type=jnp.float32,
            precision=lax.Precision.HIGHEST,
        ) + b


def _linears(x, Ws, bs):
    n, d = x.shape
    nout = len(Ws)
    grid = (n // _BLK,)
    in_specs = [pl.BlockSpec((_BLK, d), lambda i: (i, 0))]
    in_specs += [pl.BlockSpec(W.shape, lambda i: (0, 0)) for W in Ws]
    in_specs += [pl.BlockSpec((1, b.shape[1]), lambda i: (0, 0)) for b in bs]
    return pl.pallas_call(
        functools.partial(_lin_body, nout),
        grid=grid,
        in_specs=in_specs,
        out_specs=[pl.BlockSpec((_BLK, W.shape[0]), lambda i: (i, 0))
                   for W in Ws],
        out_shape=[jax.ShapeDtypeStruct((n, W.shape[0]), jnp.float32)
                   for W in Ws],
    )(x, *Ws, *bs)


WSZ = 512
GB = 64
CAP = 768
NBMAX = 16
ACC_H = 6528
ACC_T = 2 * ACC_H
ZCH = ACC_T // NS


def _ecfg(E):
    b = WSZ * -(-E // (WSZ * NS))
    return dict(E=E, b=b, nw=b // WSZ, padlen=NS * b)


ECFG = {'ww': _ecfg(200000), 'wt': _ecfg(100000), 'wd': _ecfg(200000),
        'td': _ecfg(50000), 'tt': _ecfg(50000)}

PLANS = [
    dict(out=0, npass=2, rows=12800, groups=[('ww', 0)]),
    dict(out=1, npass=1, rows=5120, groups=[('wt', 0), ('tt', ACC_H)]),
    dict(out=2, npass=4, rows=6400, groups=[('wd', 0), ('td', ACC_H)]),
]
OUT_PAD = [4 * 12800, 2 * 5120, 8 * 6400]


def _agg_body(wh_ww, wh_wt, wh_wd, wh_td, wh_tt,
              e_ww, e_wt, e_wd, e_td, e_tt,
              hw, ht, hd,
              acc_sh, deg_sh, wsem0, wsem1, dsem, zsem, nsem,
              gsem0, gsem1, ssem0, ssem1):
    pl.run_scoped(
        functools.partial(
            _agg_scoped,
            (wh_ww, wh_wt, wh_wd, wh_td, wh_tt),
            (e_ww, e_wt, e_wd, e_td, e_tt),
            (hw, ht, hd), acc_sh, deg_sh,
            (wsem0, wsem1), dsem, zsem, nsem,
            (gsem0, gsem1), (ssem0, ssem1)),
        pltpu.VMEM((CAP,), jnp.int32),
        pltpu.VMEM((CAP,), jnp.float32),
        pltpu.VMEM((NBMAX, GB), jnp.int32),
        pltpu.VMEM((2 * GB, D), jnp.float32),
        pltpu.VMEM((2 * 3 * WSZ,), jnp.int32),
        pltpu.VMEM((64,), jnp.float32),
        pltpu.VMEM((64,), jnp.float32),
        pltpu.VMEM((64,), jnp.float32),
        pltpu.VMEM((64,), jnp.float32),
        pltpu.VMEM((GB,), jnp.float32),
    )


def _agg_scoped(whs_t, ecombs_t, outs_t, acc_sh, deg_sh, wsems, dsem,
                zsem, nsem, gsems, ssems,
                csrc, cew, coffs, rows_buf, wcomb,
                degstage, recip1, recip2, dzero, ones_v):
    whs = dict(zip(['ww', 'wt', 'wd', 'td', 'tt'], whs_t))
    ecombs = dict(zip(['ww', 'wt', 'wd', 'td', 'tt'], ecombs_t))
    outs = list(outs_t)
    core = lax.axis_index("c")
    sid = lax.axis_index("s")
    i16 = lax.iota(jnp.int32, 16)

    @pl.loop(0, GB // 16)
    def _(j):
        ones_v[pl.ds(j * 16, 16)] = jnp.ones((16,), jnp.float32)

    @pl.loop(0, 4)
    def _(j):
        dzero[pl.ds(j * 16, 16)] = jnp.zeros((16,), jnp.float32)

    def g_issue(wh, bi, par):
        pltpu.async_copy(wh.at[csrc.at[pl.ds(bi * GB, GB)]],
                         rows_buf.at[pl.ds(par * GB, GB)], gsems[par])

    def g_wait(wh, bi, par):
        pltpu.make_async_copy(wh.at[csrc.at[pl.ds(bi * GB, GB)]],
                              rows_buf.at[pl.ds(par * GB, GB)],
                              gsems[par]).wait()

    def s_issue(bi, par):
        pltpu.async_copy(rows_buf.at[pl.ds(par * GB, GB)],
                         acc_sh.at[coffs.at[bi]], ssems[par], add=True)
        pltpu.async_copy(ones_v, deg_sh.at[coffs.at[bi]], dsem, add=True)

    def s_wait(par):
        pltpu.make_async_copy(rows_buf.at[pl.ds(par * GB, GB)],
                              acc_sh.at[coffs.at[0]], ssems[par]).wait()

    def scale(bi, par):
        @pl.loop(0, GB)
        def _(r):
            ewv = plsc.load_gather(
                cew, [jnp.broadcast_to(bi * GB + r, (16,)).astype(jnp.int32)])
            for j in range(8):
                sl = pl.ds(j * 16, 16)
                rows_buf[par * GB + r, sl] = rows_buf[par * GB + r, sl] * ewv

    def flush(wh, nfull):
        @pl.when(nfull > 0)
        def _():
            g_issue(wh, 0, 0)

        def pair(i, carry):
            for par in (0, 1):
                b = 2 * i + par

                @pl.when(b < nfull)
                def _():
                    g_wait(wh, b, par)

                    @pl.when(b + 1 < nfull)
                    def _():
                        @pl.when(b >= 1)
                        def _():
                            s_wait(par ^ 1)

                        g_issue(wh, b + 1, par ^ 1)

                    scale(b, par)
                    s_issue(b, par)
            return carry

        lax.fori_loop(0, lax.shift_right_logical(nfull + 1, 1), pair,
                      jnp.int32(0))

        @pl.when(nfull >= 2)
        def _():
            s_wait(0)
            s_wait(1)

        @pl.when(nfull == 1)
        def _():
            s_wait(0)

        def drd(k, carry):
            pltpu.make_async_copy(ones_v, deg_sh.at[coffs.at[0]],
                                  dsem).wait()
            return carry

        lax.fori_loop(0, nfull, drd, jnp.int32(0))

    def process(et, base, rows, acc_off):
        cfg = ECFG[et]
        wh = whs[et]
        ecomb = ecombs[et]
        nw = cfg['nw']
        blk0 = sid * nw
        dummy = acc_off + rows + sid * 8

        def w_issue(w, par):
            pltpu.async_copy(
                ecomb.at[pl.ds((blk0 + w) * 3 * WSZ, 3 * WSZ)],
                wcomb.at[pl.ds(par * 3 * WSZ, 3 * WSZ)], wsems[par])

        def w_wait(w, par):
            pltpu.make_async_copy(
                ecomb.at[pl.ds((blk0 + w) * 3 * WSZ, 3 * WSZ)],
                wcomb.at[pl.ds(par * 3 * WSZ, 3 * WSZ)], wsems[par]).wait()

        def scan_flush(w, par, n):
            w_wait(w, par)
            w_issue(w + 1, par ^ 1)
            pbase = par * 3 * WSZ

            def grp_body(g, n):
                s = wcomb[pl.ds(pbase + g * 16, 16)]
                d = wcomb[pl.ds(pbase + WSZ + g * 16, 16)]
                w_ = plsc.bitcast(wcomb[pl.ds(pbase + 2 * WSZ + g * 16, 16)],
                                  jnp.float32)
                mi = (d >= base) & (d < base + rows)
                mcnt = jnp.cumsum(mi.astype(jnp.int32))
                pos = n + mcnt - 1
                plsc.store_scatter(csrc, [pos], s, mask=mi)
                plsc.store_scatter(cew, [pos], w_, mask=mi)
                offs = d - base + acc_off
                plsc.store_scatter(
                    coffs,
                    [lax.shift_right_logical(pos, 6),
                     lax.bitwise_and(pos, 63)],
                    offs, mask=mi)
                return n + jnp.sum(mi.astype(jnp.int32))

            n = lax.fori_loop(0, WSZ // 16, grp_body, n)

            nfull = lax.shift_right_logical(n, 6)
            flush(wh, nfull)

            r = lax.bitwise_and(n, 63)
            mvbase = nfull * GB

            def mv_body(k, carry2):
                sl_from = pl.ds(mvbase + k * 16, 16)
                sl_to = pl.ds(k * 16, 16)
                csrc[sl_to] = csrc[sl_from]
                cew[sl_to] = cew[sl_from]
                idx = k * 16 + i16
                v = plsc.load_gather(
                    coffs, [jnp.broadcast_to(nfull, (16,)).astype(jnp.int32),
                            idx])
                plsc.store_scatter(
                    coffs, [jnp.zeros((16,), jnp.int32), idx], v)
                return carry2

            lax.fori_loop(0, lax.shift_right_logical(r + 15, 4), mv_body,
                          jnp.int32(0))
            return r

        w_issue(0, 0)

        def pair_body(i, n):
            n = scan_flush(2 * i, 0, n)
            n = scan_flush(2 * i + 1, 1, n)
            return n

        n = lax.fori_loop(0, nw // 2, pair_body, jnp.int32(0))
        if nw % 2:
            n = scan_flush(nw - 1, 0, n)
        w_wait(nw, nw % 2)

        start = lax.bitwise_and(n, ~jnp.int32(15))

        def pad_body(k, kcarry):
            idx = start + k * 16 + i16
            pm = idx >= n
            plsc.store_scatter(csrc, [idx], jnp.zeros((16,), jnp.int32),
                               mask=pm)
            plsc.store_scatter(cew, [idx], jnp.zeros((16,), jnp.float32),
                               mask=pm)
            plsc.store_scatter(
                coffs,
                [lax.shift_right_logical(idx, 7),
                 lax.bitwise_and(idx, 127)],
                jnp.broadcast_to(dummy, (16,)).astype(jnp.int32), mask=pm)
            return kcarry

        lax.fori_loop(0, lax.shift_right_logical(GB - start, 4), pad_body,
                      jnp.int32(0))

        nlast = lax.shift_right_logical(n + (GB - 1), 6)
        flush(wh, nlast)

    def norm_tile(out_ref, base, two, t0, sz):
        stg = rows_buf.at[pl.ds(0, sz)]
        stg2 = rows_buf.at[pl.ds(64, sz)]
        pltpu.async_copy(acc_sh.at[pl.ds(t0, sz)], stg, nsem)
        pltpu.async_copy(deg_sh.at[pl.ds(t0, sz)],
                         degstage.at[pl.ds(0, sz)], nsem)
        if two:
            pltpu.async_copy(acc_sh.at[pl.ds(ACC_H + t0, sz)], stg2, nsem)
        pltpu.make_async_copy(acc_sh.at[pl.ds(t0, sz)], stg, nsem).wait()
        pltpu.make_async_copy(deg_sh.at[pl.ds(t0, sz)],
                              degstage.at[pl.ds(0, sz)], nsem).wait()
        if two:
            pltpu.make_async_copy(acc_sh.at[pl.ds(ACC_H + t0, sz)], stg2,
                                  nsem).wait()

        @pl.loop(0, sz // 16)
        def _(k):
            dv = degstage[pl.ds(k * 16, 16)]
            recip1[pl.ds(k * 16, 16)] = 1.0 / jnp.maximum(dv, 1.0)

        if two:
            pltpu.sync_copy(deg_sh.at[pl.ds(ACC_H + t0, sz)],
                            degstage.at[pl.ds(0, sz)])

            @pl.loop(0, sz // 16)
            def _(k):
                dv = degstage[pl.ds(k * 16, 16)]
                recip2[pl.ds(k * 16, 16)] = 1.0 / jnp.maximum(dv, 1.0)

        @pl.loop(0, sz)
        def _(r):
            rs = jnp.broadcast_to(r, (16,)).astype(jnp.int32)
            g1 = plsc.load_gather(recip1, [rs])
            if two:
                g2 = plsc.load_gather(recip2, [rs])
            for j in range(8):
                sl = pl.ds(j * 16, 16)
                v = rows_buf[r, sl] * g1
                if two:
                    v = v + rows_buf[64 + r, sl] * g2
                rows_buf[r, sl] = v

        pltpu.sync_copy(stg, out_ref.at[pl.ds(base + t0, sz)])

    def normalize_out(out_ref, base, rows, two):
        rows_n = rows // NS
        off0 = sid * rows_n

        def tile_body(t, carry):
            norm_tile(out_ref, base, two, off0 + t * 64, 64)
            return carry

        lax.fori_loop(0, rows_n // 64, tile_body, jnp.int32(0))
        if rows_n % 64:
            norm_tile(out_ref, base, two, off0 + (rows_n // 64) * 64,
                      rows_n % 64)

    def run_plan(plan):
        rows = plan['rows']
        out_ref = outs[plan['out']]

        def pass_body(p, pcarry):
            base = (core * plan['npass'] + p) * rows

            @pl.loop(0, GB)
            def _(r):
                for j in range(8):
                    rows_buf[r, pl.ds(j * 16, 16)] = jnp.zeros(
                        (16,), jnp.float32)

            zb = sid * ZCH
            nz = ZCH // 64
            zr = ZCH - nz * 64

            def z_issue(k, carry):
                pltpu.async_copy(rows_buf.at[pl.ds(0, 64)],
                                 acc_sh.at[pl.ds(zb + k * 64, 64)], zsem)
                pltpu.async_copy(dzero, deg_sh.at[pl.ds(zb + k * 64, 64)],
                                 zsem)
                return carry

            lax.fori_loop(0, nz, z_issue, jnp.int32(0))
            pltpu.async_copy(rows_buf.at[pl.ds(0, zr)],
                             acc_sh.at[pl.ds(zb + nz * 64, zr)], zsem)
            pltpu.async_copy(dzero.at[pl.ds(0, zr - 16)],
                             deg_sh.at[pl.ds(zb + nz * 64, zr - 16)], zsem)
            pltpu.async_copy(dzero.at[pl.ds(0, 16)],
                             deg_sh.at[pl.ds(zb + nz * 64 + zr - 16, 16)],
                             zsem)

            def z_drain(k, carry):
                pltpu.make_async_copy(
                    rows_buf.at[pl.ds(0, 64)],
                    acc_sh.at[pl.ds(zb + k * 64, 64)], zsem).wait()
                pltpu.make_async_copy(
                    dzero, deg_sh.at[pl.ds(zb + k * 64, 64)], zsem).wait()
                return carry

            lax.fori_loop(0, nz, z_drain, jnp.int32(0))
            pltpu.make_async_copy(rows_buf.at[pl.ds(0, zr)],
                                  acc_sh.at[pl.ds(zb + nz * 64, zr)],
                                  zsem).wait()
            pltpu.make_async_copy(dzero.at[pl.ds(0, zr - 16)],
                                  deg_sh.at[pl.ds(zb + nz * 64, zr - 16)],
                                  zsem).wait()
            pltpu.make_async_copy(
                dzero.at[pl.ds(0, 16)],
                deg_sh.at[pl.ds(zb + nz * 64 + zr - 16, 16)], zsem).wait()

            plsc.subcore_barrier()
            for (et, acc_off) in plan['groups']:
                process(et, base, rows, acc_off)
            plsc.subcore_barrier()
            normalize_out(out_ref, base, rows, len(plan['groups']) == 2)
            plsc.subcore_barrier()
            return pcarry

        lax.fori_loop(0, plan['npass'], pass_body, jnp.int32(0))

    for plan in PLANS:
        run_plan(plan)


def _aggregate(whs, ecombs):
    mesh = plsc.VectorSubcoreMesh(core_axis_name="c", subcore_axis_name="s")
    out_type = [jax.ShapeDtypeStruct((OUT_PAD[i], D), jnp.float32)
                for i in range(3)]
    scratch = [
        pltpu.VMEM_SHARED((ACC_T, D), jnp.float32),
        pltpu.VMEM_SHARED((ACC_T,), jnp.float32),
        pltpu.SemaphoreType.DMA,
        pltpu.SemaphoreType.DMA,
        pltpu.SemaphoreType.DMA,
        pltpu.SemaphoreType.DMA,
        pltpu.SemaphoreType.DMA,
        pltpu.SemaphoreType.DMA,
        pltpu.SemaphoreType.DMA,
        pltpu.SemaphoreType.DMA,
        pltpu.SemaphoreType.DMA,
    ]
    cp = pltpu.CompilerParams()
    if "needs_layout_passes" in pltpu.CompilerParams.__dataclass_fields__:
        cp = dataclasses.replace(cp, needs_layout_passes=False)
    fn = pl.kernel(_agg_body, out_type=out_type, mesh=mesh,
                   scratch_types=scratch, compiler_params=cp)
    return fn(*whs, *ecombs)


def kernel(feat_word, feat_topic, effect, src_ww, dst_ww, ew_ww, src_wt, dst_wt, ew_wt, src_wd, dst_wd, ew_wd, src_td, dst_td, ew_td, src_tt, dst_tt, ew_tt, W_ww, b_ww, W_wt, b_wt, W_wd, b_wd, W_td, b_td, W_tt, b_tt, W_td_cau, b_td_cau, W_td_noi, b_td_noi, W_tt_cau, b_tt_cau, W_tt_noi, b_tt_noi, W_td_cau_trans, W_td_noi_trans, W_tt_cau_trans, W_tt_noi_trans):
    Nw = feat_word.shape[0]
    Nt = feat_topic.shape[0]
    Nd = 50000

    Wh_ww, Wh_wt, Wh_wd = _linears(
        feat_word, [W_ww, W_wt, W_wd],
        [b_ww.reshape(1, -1), b_wt.reshape(1, -1), b_wd.reshape(1, -1)])
    Wh_td, Wh_tt = _linears(
        feat_topic, [W_td, W_tt],
        [b_td.reshape(1, -1), b_tt.reshape(1, -1)])

    def interleave(et, src, dst, ew):
        cfg = ECFG[et]
        p = cfg['padlen'] - cfg['E']
        srcw = jnp.pad(src, (0, p)).reshape(-1, WSZ)
        dstw = jnp.pad(dst, (0, p),
                       constant_values=jnp.int32(2**30)).reshape(-1, WSZ)
        ewb = lax.bitcast_convert_type(
            jnp.pad(ew.reshape(-1), (0, p)), jnp.int32).reshape(-1, WSZ)
        flat = jnp.stack([srcw, dstw, ewb], axis=1).reshape(-1)
        return jnp.pad(flat, (0, 3 * WSZ))

    ecombs = [interleave('ww', src_ww, dst_ww, ew_ww),
              interleave('wt', src_wt, dst_wt, ew_wt),
              interleave('wd', src_wd, dst_wd, ew_wd),
              interleave('td', src_td, dst_td, ew_td),
              interleave('tt', src_tt, dst_tt, ew_tt)]

    hw, ht, hd = _aggregate([Wh_ww, Wh_wt, Wh_wd, Wh_td, Wh_tt], ecombs)
    return (hw[:Nw], ht[:Nt], hd[:Nd])
